# baseline (speedup 1.0000x reference)
import numpy as np
import jax
import jax.numpy as jnp
from jax.experimental import pallas as pl

_N = 4096
_K = 2048
_EBNO_DB = 2.0
_THRESH = 0.9
_BATCH = 1024
_ROWS = 256
_QG = _N // 512


def _code_construction():
    z = np.array([0.5], dtype=np.float64)
    while z.size < _N:
        z = np.concatenate([2.0 * z - z * z, z * z])
    info_set = np.sort(np.argsort(z)[:_K])
    return info_set, z


_SIGMA = float(np.sqrt(1.0 / (2.0 * (_K / _N) * 10.0 ** (_EBNO_DB / 10.0))))


def _build_constants():
    info_set, z = _code_construction()
    l = np.arange(512)
    g512 = ((l[:, None] & ~l[None, :]) == 0)

    k0 = np.searchsorted(info_set, np.arange(_QG) * 512)
    k1 = np.searchsorted(info_set, (np.arange(_QG) + 1) * 512)
    enc_off = np.zeros((_QG,), np.int64)
    enc_z = np.zeros((_QG, 512, 512), np.float32)
    for q in range(_QG):
        a = min(int(k0[q]), _K - 512)
        enc_off[q] = a
        for k in range(int(k0[q]), int(k1[q])):
            ll = int(info_set[k]) % 512
            enc_z[q, k - a, :] = g512[:, ll]

    m256 = g512.T.astype(np.float32)

    r = 1.0 - z
    t = 1.8 - r
    tau = np.full((_N,), np.inf)
    fin = (t > 0.0) & (t < 1.0)
    tau[fin] = (_SIGMA * _SIGMA / 2.0) * np.log(t[fin] / (1.0 - t[fin]))
    tau[t <= 0.0] = -np.inf
    frozen = np.ones((_N,), bool)
    frozen[info_set] = False
    tau[frozen] = np.inf
    consts = np.zeros((8, _N), np.float32)
    consts[0, :] = tau.astype(np.float32)
    return enc_off, enc_z, m256, consts


_ENC_OFF, _ENC_Z_NP, _M256_NP, _CONSTS_NP = _build_constants()


def _polar_metrics_kernel(bits_ref, noise_ref, ez_ref, m_ref, c_ref, out_ref):
    i = pl.program_id(0)

    @pl.when(i == 0)
    def _init():
        out_ref[...] = jnp.zeros_like(out_ref)

    bits_bf = bits_ref[...].astype(jnp.float8_e4m3fn)
    m256 = m_ref[...]

    w = []
    for q in range(_QG):
        a = int(_ENC_OFF[q])
        w.append(jnp.dot(bits_bf[:, a:a + 512], ez_ref[q],
                         preferred_element_type=jnp.float32))

    for s in (1, 2, 4):
        for q in range(_QG):
            if q & s == 0:
                w[q] = w[q] + w[q + s]

    e = [None] * _QG
    masks = [None] * _QG
    for q in range(_QG):
        x_int = jnp.bitwise_and(w[q].astype(jnp.int32), 1)
        nbits = jax.lax.bitcast_convert_type(
            noise_ref[:, 512 * q:512 * (q + 1)], jnp.int32)
        sflip = jax.lax.bitcast_convert_type(
            jnp.bitwise_xor(nbits, x_int << 31), jnp.float32)
        v = 1.0 + _SIGMA * sflip
        tau = c_ref[0:1, 512 * q:512 * (q + 1)]
        masks[q] = (jnp.abs(v) >= tau).astype(jnp.float32)
        e[q] = (v < 0.0).astype(jnp.float8_e4m3fn)

    t = [jnp.dot(e[q], m256, preferred_element_type=jnp.float32)
         for q in range(_QG)]
    for s in (1, 2, 4):
        for q in range(_QG):
            if q & s == 0:
                t[q] = t[q] + t[q + s]

    acc_m = jnp.zeros((_ROWS, 512), jnp.float32)
    acc_em = jnp.zeros((_ROWS, 512), jnp.float32)
    for q in range(_QG):
        err_q = jnp.bitwise_and(t[q].astype(jnp.int32), 1).astype(jnp.float32)
        acc_m = acc_m + masks[q]
        acc_em = acc_em + err_q * masks[q]
    s_mask = jnp.sum(acc_m)
    s_em = jnp.sum(acc_em)
    row_em = jnp.sum(acc_em, axis=1, keepdims=True)
    fer_cnt = jnp.sum((row_em > 0.0).astype(jnp.float32))

    lane = jax.lax.broadcasted_iota(jnp.int32, (8, 128), 1)
    partial = (jnp.where(lane == 0, s_mask, 0.0)
               + jnp.where(lane == 1, s_em, 0.0)
               + jnp.where(lane == 2, fer_cnt, 0.0))
    out_ref[...] = out_ref[...] + partial


def kernel(info_bits, noise):
    ez = jnp.asarray(_ENC_Z_NP, jnp.float8_e4m3fn)
    m256 = jnp.asarray(_M256_NP, jnp.float8_e4m3fn)
    consts = jnp.asarray(_CONSTS_NP)
    grid = (_BATCH // _ROWS,)
    sums = pl.pallas_call(
        _polar_metrics_kernel,
        grid=grid,
        in_specs=[
            pl.BlockSpec((_ROWS, _K), lambda i: (i, 0)),
            pl.BlockSpec((_ROWS, _N), lambda i: (i, 0)),
            pl.BlockSpec((_QG, 512, 512), lambda i: (0, 0, 0)),
            pl.BlockSpec((512, 512), lambda i: (0, 0)),
            pl.BlockSpec((8, _N), lambda i: (0, 0)),
        ],
        out_specs=pl.BlockSpec((8, 128), lambda i: (0, 0)),
        out_shape=jax.ShapeDtypeStruct((8, 128), jnp.float32),
    )(info_bits, noise, ez, m256, consts)
    s_mask = sums[0, 0]
    s_em = sums[0, 1]
    fer_cnt = sums[0, 2]
    b = jnp.float32(_BATCH)
    ber = s_em / jnp.maximum(s_mask, 1.0)
    fer = fer_cnt / b
    rate = s_mask / b
    return (ber, fer, rate)


# speedup vs baseline: 1.0154x; 1.0154x over previous
import numpy as np
import jax
import jax.numpy as jnp
from jax.experimental import pallas as pl

_N = 4096
_K = 2048
_EBNO_DB = 2.0
_THRESH = 0.9
_BATCH = 1024
_ROWS = 128
_QG = _N // 256


def _code_construction():
    z = np.array([0.5], dtype=np.float64)
    while z.size < _N:
        z = np.concatenate([2.0 * z - z * z, z * z])
    info_set = np.sort(np.argsort(z)[:_K])
    return info_set, z


_SIGMA = float(np.sqrt(1.0 / (2.0 * (_K / _N) * 10.0 ** (_EBNO_DB / 10.0))))


def _build_constants():
    info_set, z = _code_construction()
    l = np.arange(256)
    g256 = ((l[:, None] & ~l[None, :]) == 0)

    k0 = np.searchsorted(info_set, np.arange(_QG) * 256)
    k1 = np.searchsorted(info_set, (np.arange(_QG) + 1) * 256)
    enc_off = np.zeros((_QG,), np.int64)
    enc_z = np.zeros((_QG, 256, 256), np.float32)
    for q in range(_QG):
        a = min(int(k0[q]), _K - 256)
        enc_off[q] = a
        for k in range(int(k0[q]), int(k1[q])):
            ll = int(info_set[k]) % 256
            enc_z[q, k - a, :] = g256[:, ll]

    m256 = g256.T.astype(np.float32)

    r = 1.0 - z
    t = 1.8 - r
    tau = np.full((_N,), np.inf)
    fin = (t > 0.0) & (t < 1.0)
    tau[fin] = (_SIGMA * _SIGMA / 2.0) * np.log(t[fin] / (1.0 - t[fin]))
    tau[t <= 0.0] = -np.inf
    frozen = np.ones((_N,), bool)
    frozen[info_set] = False
    tau[frozen] = np.inf
    consts = np.zeros((8, _N), np.float32)
    consts[0, :] = tau.astype(np.float32)
    return enc_off, enc_z, m256, consts


_ENC_OFF, _ENC_Z_NP, _M256_NP, _CONSTS_NP = _build_constants()


def _polar_metrics_kernel(bits_ref, noise_ref, ez_ref, m_ref, c_ref, out_ref):
    i = pl.program_id(0)

    @pl.when(i == 0)
    def _init():
        out_ref[...] = jnp.zeros_like(out_ref)

    bits_bf = bits_ref[...].astype(jnp.float8_e4m3fn)
    m256 = m_ref[...]

    w = []
    for q in range(_QG):
        a = int(_ENC_OFF[q])
        w.append(jnp.dot(bits_bf[:, a:a + 256], ez_ref[q],
                         preferred_element_type=jnp.float32))

    for s in (1, 2, 4, 8):
        for q in range(_QG):
            if q & s == 0:
                w[q] = w[q] + w[q + s]

    e = [None] * _QG
    masks = [None] * _QG
    for q in range(_QG):
        x_int = jnp.bitwise_and(w[q].astype(jnp.int32), 1)
        nbits = jax.lax.bitcast_convert_type(
            noise_ref[:, 256 * q:256 * (q + 1)], jnp.int32)
        sflip = jax.lax.bitcast_convert_type(
            jnp.bitwise_xor(nbits, x_int << 31), jnp.float32)
        v = 1.0 + _SIGMA * sflip
        tau = c_ref[0:1, 256 * q:256 * (q + 1)]
        masks[q] = (jnp.abs(v) >= tau).astype(jnp.float32)
        e[q] = (v < 0.0).astype(jnp.float8_e4m3fn)

    t = [jnp.dot(e[q], m256, preferred_element_type=jnp.float32)
         for q in range(_QG)]
    for s in (1, 2, 4, 8):
        for q in range(_QG):
            if q & s == 0:
                t[q] = t[q] + t[q + s]

    acc_m = jnp.zeros((_ROWS, 256), jnp.float32)
    acc_em = jnp.zeros((_ROWS, 256), jnp.float32)
    for q in range(_QG):
        err_q = jnp.bitwise_and(t[q].astype(jnp.int32), 1).astype(jnp.float32)
        acc_m = acc_m + masks[q]
        acc_em = acc_em + err_q * masks[q]
    s_mask = jnp.sum(acc_m)
    s_em = jnp.sum(acc_em)
    row_em = jnp.sum(acc_em, axis=1, keepdims=True)
    fer_cnt = jnp.sum((row_em > 0.0).astype(jnp.float32))

    lane = jax.lax.broadcasted_iota(jnp.int32, (8, 128), 1)
    partial = (jnp.where(lane == 0, s_mask, 0.0)
               + jnp.where(lane == 1, s_em, 0.0)
               + jnp.where(lane == 2, fer_cnt, 0.0))
    out_ref[...] = out_ref[...] + partial


def kernel(info_bits, noise):
    ez = jnp.asarray(_ENC_Z_NP, jnp.float8_e4m3fn)
    m256 = jnp.asarray(_M256_NP, jnp.float8_e4m3fn)
    consts = jnp.asarray(_CONSTS_NP)
    grid = (_BATCH // _ROWS,)
    sums = pl.pallas_call(
        _polar_metrics_kernel,
        grid=grid,
        in_specs=[
            pl.BlockSpec((_ROWS, _K), lambda i: (i, 0)),
            pl.BlockSpec((_ROWS, _N), lambda i: (i, 0)),
            pl.BlockSpec((_QG, 256, 256), lambda i: (0, 0, 0)),
            pl.BlockSpec((256, 256), lambda i: (0, 0)),
            pl.BlockSpec((8, _N), lambda i: (0, 0)),
        ],
        out_specs=pl.BlockSpec((8, 128), lambda i: (0, 0)),
        out_shape=jax.ShapeDtypeStruct((8, 128), jnp.float32),
    )(info_bits, noise, ez, m256, consts)
    s_mask = sums[0, 0]
    s_em = sums[0, 1]
    fer_cnt = sums[0, 2]
    b = jnp.float32(_BATCH)
    ber = s_em / jnp.maximum(s_mask, 1.0)
    fer = fer_cnt / b
    rate = s_mask / b
    return (ber, fer, rate)


# speedup vs baseline: 1.0320x; 1.0163x over previous
import numpy as np
import jax
import jax.numpy as jnp
from jax.experimental import pallas as pl

_N = 4096
_K = 2048
_EBNO_DB = 2.0
_THRESH = 0.9
_BATCH = 1024
_ROWS = 256
_QG = _N // 256


def _code_construction():
    z = np.array([0.5], dtype=np.float64)
    while z.size < _N:
        z = np.concatenate([2.0 * z - z * z, z * z])
    info_set = np.sort(np.argsort(z)[:_K])
    return info_set, z


_SIGMA = float(np.sqrt(1.0 / (2.0 * (_K / _N) * 10.0 ** (_EBNO_DB / 10.0))))


def _build_constants():
    info_set, z = _code_construction()
    l = np.arange(256)
    g256 = ((l[:, None] & ~l[None, :]) == 0)

    k0 = np.searchsorted(info_set, np.arange(_QG) * 256)
    k1 = np.searchsorted(info_set, (np.arange(_QG) + 1) * 256)
    enc_off = np.zeros((_QG,), np.int64)
    enc_z = np.zeros((_QG, 256, 256), np.float32)
    for q in range(_QG):
        a = min(int(k0[q]), _K - 256)
        enc_off[q] = a
        for k in range(int(k0[q]), int(k1[q])):
            ll = int(info_set[k]) % 256
            enc_z[q, k - a, :] = g256[:, ll]

    m256 = g256.T.astype(np.float32)

    r = 1.0 - z
    t = 1.8 - r
    tau = np.full((_N,), np.inf)
    fin = (t > 0.0) & (t < 1.0)
    tau[fin] = (_SIGMA * _SIGMA / 2.0) * np.log(t[fin] / (1.0 - t[fin]))
    tau[t <= 0.0] = -np.inf
    frozen = np.ones((_N,), bool)
    frozen[info_set] = False
    tau[frozen] = np.inf
    consts = np.zeros((8, _N), np.float32)
    consts[0, :] = tau.astype(np.float32)
    return enc_off, enc_z, m256, consts


_ENC_OFF, _ENC_Z_NP, _M256_NP, _CONSTS_NP = _build_constants()


def _polar_metrics_kernel(bits_ref, noise_ref, ez_ref, m_ref, c_ref, out_ref):
    i = pl.program_id(0)

    @pl.when(i == 0)
    def _init():
        out_ref[...] = jnp.zeros_like(out_ref)

    bits_bf = bits_ref[...].astype(jnp.float8_e4m3fn)
    m256 = m_ref[...]

    w = []
    for q in range(_QG):
        a = int(_ENC_OFF[q])
        w.append(jnp.dot(bits_bf[:, a:a + 256], ez_ref[q],
                         preferred_element_type=jnp.float32))

    for s in (1, 2, 4, 8):
        for q in range(_QG):
            if q & s == 0:
                w[q] = w[q] + w[q + s]

    e = [None] * _QG
    masks = [None] * _QG
    for q in range(_QG):
        x_int = jnp.bitwise_and(w[q].astype(jnp.int32), 1)
        nbits = jax.lax.bitcast_convert_type(
            noise_ref[:, 256 * q:256 * (q + 1)], jnp.int32)
        sflip = jax.lax.bitcast_convert_type(
            jnp.bitwise_xor(nbits, x_int << 31), jnp.float32)
        v = 1.0 + _SIGMA * sflip
        tau = c_ref[0:1, 256 * q:256 * (q + 1)]
        masks[q] = (jnp.abs(v) >= tau).astype(jnp.float32)
        e[q] = (v < 0.0).astype(jnp.float8_e4m3fn)

    t = [jnp.dot(e[q], m256, preferred_element_type=jnp.float32)
         for q in range(_QG)]
    for s in (1, 2, 4, 8):
        for q in range(_QG):
            if q & s == 0:
                t[q] = t[q] + t[q + s]

    acc_m = jnp.zeros((_ROWS, 256), jnp.float32)
    acc_em = jnp.zeros((_ROWS, 256), jnp.float32)
    for q in range(_QG):
        err_q = jnp.bitwise_and(t[q].astype(jnp.int32), 1).astype(jnp.float32)
        acc_m = acc_m + masks[q]
        acc_em = acc_em + err_q * masks[q]
    s_mask = jnp.sum(acc_m)
    s_em = jnp.sum(acc_em)
    row_em = jnp.sum(acc_em, axis=1, keepdims=True)
    fer_cnt = jnp.sum((row_em > 0.0).astype(jnp.float32))

    lane = jax.lax.broadcasted_iota(jnp.int32, (8, 128), 1)
    partial = (jnp.where(lane == 0, s_mask, 0.0)
               + jnp.where(lane == 1, s_em, 0.0)
               + jnp.where(lane == 2, fer_cnt, 0.0))
    out_ref[...] = out_ref[...] + partial


def kernel(info_bits, noise):
    ez = jnp.asarray(_ENC_Z_NP, jnp.float8_e4m3fn)
    m256 = jnp.asarray(_M256_NP, jnp.float8_e4m3fn)
    consts = jnp.asarray(_CONSTS_NP)
    grid = (_BATCH // _ROWS,)
    sums = pl.pallas_call(
        _polar_metrics_kernel,
        grid=grid,
        in_specs=[
            pl.BlockSpec((_ROWS, _K), lambda i: (i, 0)),
            pl.BlockSpec((_ROWS, _N), lambda i: (i, 0)),
            pl.BlockSpec((_QG, 256, 256), lambda i: (0, 0, 0)),
            pl.BlockSpec((256, 256), lambda i: (0, 0)),
            pl.BlockSpec((8, _N), lambda i: (0, 0)),
        ],
        out_specs=pl.BlockSpec((8, 128), lambda i: (0, 0)),
        out_shape=jax.ShapeDtypeStruct((8, 128), jnp.float32),
    )(info_bits, noise, ez, m256, consts)
    s_mask = sums[0, 0]
    s_em = sums[0, 1]
    fer_cnt = sums[0, 2]
    b = jnp.float32(_BATCH)
    ber = s_em / jnp.maximum(s_mask, 1.0)
    fer = fer_cnt / b
    rate = s_mask / b
    return (ber, fer, rate)
